# baseline (device time: 24238 ns/iter reference)
import os

import jax
import jax.numpy as jnp
from jax import lax
from jax.experimental import pallas as pl
from jax.experimental.pallas import tpu as pltpu

N_DEV = 8

ABLATE_COMM = os.environ.get("ABL_COMM") == "1"


def kernel(x, w_mat):
    k_glob, k_blk = x.shape
    _, n = w_mat.shape
    m_blk = k_glob // N_DEV

    def body(x_hbm, w_hbm, out_hbm, xv_ref, xb_ref, comm_ref, wb_ref, ov_ref,
             x_sem, send_sems, recv_sems, w_sems, out_sem):
        my = lax.axis_index("i")

        xdma = pltpu.make_async_copy(x_hbm, xv_ref, x_sem)
        xdma.start()

        wdmas = []
        for k in range(N_DEV):
            src = (my - k) % N_DEV
            wdma = pltpu.make_async_copy(
                w_hbm.at[pl.ds(src * m_blk, m_blk), :],
                wb_ref.at[k],
                w_sems.at[k],
            )
            wdma.start()
            wdmas.append(wdma)

        barrier_sem = pltpu.get_barrier_semaphore()
        for k in range(1, N_DEV):
            pl.semaphore_signal(
                barrier_sem, inc=1,
                device_id=((my + k) % N_DEV,),
                device_id_type=pl.DeviceIdType.MESH,
            )
        pl.semaphore_wait(barrier_sem, N_DEV - 1)

        xdma.wait()
        xb_ref[:, :] = xv_ref[:, :].astype(jnp.bfloat16)

        rdmas = []
        if not ABLATE_COMM:
            for k in range(1, N_DEV):
                dst = (my + k) % N_DEV
                rdma = pltpu.make_async_remote_copy(
                    src_ref=xb_ref.at[pl.ds(dst * m_blk, m_blk), :],
                    dst_ref=comm_ref.at[k - 1],
                    send_sem=send_sems.at[k - 1],
                    recv_sem=recv_sems.at[k - 1],
                    device_id=(dst,),
                    device_id_type=pl.DeviceIdType.MESH,
                )
                rdma.start()
                rdmas.append(rdma)

        wdmas[0].wait()
        acc = jnp.dot(
            xb_ref[pl.ds(my * m_blk, m_blk), :],
            wb_ref[0].astype(jnp.bfloat16),
            preferred_element_type=jnp.float32,
        )
        for k in range(1, N_DEV):
            if not ABLATE_COMM:
                rdmas[k - 1].wait_recv()
                blk = comm_ref[k - 1]
            else:
                src = (my - k) % N_DEV
                blk = xb_ref[pl.ds(src * m_blk, m_blk), :]
            wdmas[k].wait()
            acc = acc + jnp.dot(
                blk,
                wb_ref[k].astype(jnp.bfloat16),
                preferred_element_type=jnp.float32,
            )

        c = 0.7978845608028654
        ov_ref[:, :] = 0.5 * acc * (
            1.0 + jnp.tanh(c * (acc + 0.044715 * acc * acc * acc))
        )
        odma = pltpu.make_async_copy(ov_ref, out_hbm, out_sem)
        odma.start()
        odma.wait()

        if not ABLATE_COMM:
            for k in range(1, N_DEV):
                rdmas[k - 1].wait_send()

    return pl.pallas_call(
        body,
        out_shape=jax.ShapeDtypeStruct((m_blk, n), jnp.float32),
        in_specs=[
            pl.BlockSpec(memory_space=pltpu.MemorySpace.HBM),
            pl.BlockSpec(memory_space=pltpu.MemorySpace.HBM),
        ],
        out_specs=pl.BlockSpec(memory_space=pltpu.MemorySpace.HBM),
        scratch_shapes=[
            pltpu.VMEM((k_glob, k_blk), jnp.float32),
            pltpu.VMEM((k_glob, k_blk), jnp.bfloat16),
            pltpu.VMEM((N_DEV - 1, m_blk, k_blk), jnp.bfloat16),
            pltpu.VMEM((N_DEV, m_blk, n), jnp.float32),
            pltpu.VMEM((m_blk, n), jnp.float32),
            pltpu.SemaphoreType.DMA,
            pltpu.SemaphoreType.DMA((N_DEV - 1,)),
            pltpu.SemaphoreType.DMA((N_DEV - 1,)),
            pltpu.SemaphoreType.DMA((N_DEV,)),
            pltpu.SemaphoreType.DMA,
        ],
        compiler_params=pltpu.CompilerParams(collective_id=0),
    )(x, w_mat)


# device time: 10880 ns/iter; 2.2278x vs baseline; 2.2278x over previous
import os

import jax
import jax.numpy as jnp
from jax import lax
from jax.experimental import pallas as pl
from jax.experimental.pallas import tpu as pltpu

N_DEV = 8

ABLATE_COMM = os.environ.get("ABL_COMM") == "1"


def kernel(x, w_mat):
    k_glob, k_blk = x.shape
    _, n = w_mat.shape
    m_blk = k_glob // N_DEV

    def body(x_hbm, w_hbm, out_hbm, xv_ref, xb_ref, comm_ref, wb_ref, ov_ref,
             x_sem, send_sems, recv_sems, w_sems, out_sem):
        my = lax.axis_index("i")

        xdma = pltpu.make_async_copy(x_hbm, xv_ref, x_sem)
        xdma.start()

        wdmas = []
        for k in range(N_DEV):
            src = (my - k) % N_DEV
            wdma = pltpu.make_async_copy(
                w_hbm.at[pl.ds(src * m_blk, m_blk), :],
                wb_ref.at[k],
                w_sems.at[k],
            )
            wdma.start()
            wdmas.append(wdma)

        barrier_sem = pltpu.get_barrier_semaphore()
        for k in range(1, N_DEV):
            pl.semaphore_signal(
                barrier_sem, inc=1,
                device_id=((my + k) % N_DEV,),
                device_id_type=pl.DeviceIdType.MESH,
            )
        pl.semaphore_wait(barrier_sem, N_DEV - 1)

        xdma.wait()
        xb_ref[:, :] = xv_ref[:, :].astype(jnp.bfloat16)

        rdmas = []
        if not ABLATE_COMM:
            for k in range(1, N_DEV):
                dst = (my + k) % N_DEV
                rdma = pltpu.make_async_remote_copy(
                    src_ref=xb_ref.at[pl.ds(dst * m_blk, m_blk), :],
                    dst_ref=comm_ref.at[k - 1],
                    send_sem=send_sems.at[k - 1],
                    recv_sem=recv_sems.at[k - 1],
                    device_id=(dst,),
                    device_id_type=pl.DeviceIdType.MESH,
                )
                rdma.start()
                rdmas.append(rdma)

        wdmas[0].wait()
        acc = jnp.dot(
            xb_ref[pl.ds(my * m_blk, m_blk), :],
            wb_ref[0].astype(jnp.bfloat16),
            preferred_element_type=jnp.float32,
        )
        for k in range(1, N_DEV):
            if not ABLATE_COMM:
                rdmas[k - 1].wait_recv()
                blk = comm_ref[k - 1]
            else:
                src = (my - k) % N_DEV
                blk = xb_ref[pl.ds(src * m_blk, m_blk), :]
            wdmas[k].wait()
            acc = acc + jnp.dot(
                blk,
                wb_ref[k].astype(jnp.bfloat16),
                preferred_element_type=jnp.float32,
            )

        c = 0.7978845608028654
        ov_ref[:, :] = 0.5 * acc * (
            1.0 + jnp.tanh(c * (acc + 0.044715 * acc * acc * acc))
        )
        odma = pltpu.make_async_copy(ov_ref, out_hbm, out_sem)
        odma.start()
        odma.wait()

        if not ABLATE_COMM:
            for k in range(1, N_DEV):
                rdmas[k - 1].wait_send()

    return pl.pallas_call(
        body,
        out_shape=jax.ShapeDtypeStruct((m_blk, n), jnp.float32),
        in_specs=[
            pl.BlockSpec(memory_space=pltpu.MemorySpace.HBM),
            pl.BlockSpec(memory_space=pltpu.MemorySpace.HBM),
        ],
        out_specs=pl.BlockSpec(memory_space=pltpu.MemorySpace.HBM),
        scratch_shapes=[
            pltpu.VMEM((k_glob, k_blk), jnp.float32),
            pltpu.VMEM((k_glob, k_blk), jnp.bfloat16),
            pltpu.VMEM((N_DEV - 1, m_blk, k_blk), jnp.bfloat16),
            pltpu.VMEM((N_DEV, m_blk, n), jnp.float32),
            pltpu.VMEM((m_blk, n), jnp.float32),
            pltpu.SemaphoreType.DMA,
            pltpu.SemaphoreType.DMA((N_DEV - 1,)),
            pltpu.SemaphoreType.DMA((N_DEV - 1,)),
            pltpu.SemaphoreType.DMA((N_DEV,)),
            pltpu.SemaphoreType.DMA,
        ],
        compiler_params=pltpu.CompilerParams(collective_id=0),
    )(
        pltpu.with_memory_space_constraint(x, pltpu.MemorySpace.HBM),
        pltpu.with_memory_space_constraint(w_mat, pltpu.MemorySpace.HBM),
    )
